# baseline (speedup 1.0000x reference)
import jax
import jax.numpy as jnp
from jax import lax
from jax.experimental import pallas as pl
from jax.experimental.pallas import tpu as pltpu

N = 320000
F_IN = 128
H = 128
C = 64
B = 10000

R = 6400
NBLK = N // R
G = 640
NCH = R // G
W = 32
ACC_ROWS = B + 2 * W


def _body(s0_ref, idx_ref, sc_ref, x_ref, w1_ref, w2_ref, w3_ref, w4_ref,
          out_ref, acc_ref):
    pid = pl.program_id(0)

    @pl.when(pid == 0)
    def _init():
        acc_ref[...] = jnp.zeros((ACC_ROWS, H), jnp.float32)

    w1 = w1_ref[...].astype(jnp.bfloat16)
    seg = idx_ref[0]
    sc = sc_ref[0]
    iota = lax.broadcasted_iota(jnp.int32, (W, G), 0)

    for c in range(NCH):
        x_c = x_ref[c * G:(c + 1) * G, :].astype(jnp.bfloat16)
        h_c = jnp.maximum(
            jnp.dot(x_c, w1, preferred_element_type=jnp.float32), 0.0
        ).astype(jnp.bfloat16)
        seg_c = seg[:, c * G:(c + 1) * G]
        sc_c = sc[:, c * G:(c + 1) * G]
        s0 = s0_ref[pid * NCH + c]
        local = seg_c - s0
        nwin = jnp.max(local) // W + 1

        def win(k, carry, local=local, sc_c=sc_c, h_c=h_c, s0=s0):
            base = k * W
            oh = jnp.where(local == base + iota, sc_c,
                           0.0).astype(jnp.bfloat16)
            contrib = lax.dot_general(oh, h_c, (((1,), (0,)), ((), ())),
                                      preferred_element_type=jnp.float32)
            acc_ref[pl.ds(s0 + base, W), :] += contrib
            return carry

        lax.fori_loop(0, nwin, win, 0)

    @pl.when(pid == NBLK - 1)
    def _final():
        w23 = jnp.dot(w2_ref[...].astype(jnp.bfloat16),
                      w3_ref[...].astype(jnp.bfloat16),
                      preferred_element_type=jnp.float32).astype(jnp.bfloat16)
        p = acc_ref[0:B, :].astype(jnp.bfloat16)
        h2 = jnp.maximum(
            jnp.dot(p, w23, preferred_element_type=jnp.float32), 0.0
        ).astype(jnp.bfloat16)
        out_ref[...] = jnp.dot(h2, w4_ref[...].astype(jnp.bfloat16),
                               preferred_element_type=jnp.float32)


def kernel(X, ppr_scores, ppr_idx, W1, W2, W3, W4):
    s0s = ppr_idx[::G]
    idx3 = ppr_idx.reshape(NBLK, 1, R)
    sc3 = ppr_scores.reshape(NBLK, 1, R)

    grid_spec = pltpu.PrefetchScalarGridSpec(
        num_scalar_prefetch=1,
        grid=(NBLK,),
        in_specs=[
            pl.BlockSpec((1, 1, R), lambda i, s0s: (i, 0, 0)),
            pl.BlockSpec((1, 1, R), lambda i, s0s: (i, 0, 0)),
            pl.BlockSpec((R, F_IN), lambda i, s0s: (i, 0)),
            pl.BlockSpec((F_IN, H), lambda i, s0s: (0, 0)),
            pl.BlockSpec((H, H), lambda i, s0s: (0, 0)),
            pl.BlockSpec((H, H), lambda i, s0s: (0, 0)),
            pl.BlockSpec((H, C), lambda i, s0s: (0, 0)),
        ],
        out_specs=pl.BlockSpec((B, C), lambda i, s0s: (0, 0)),
        scratch_shapes=[pltpu.VMEM((ACC_ROWS, H), jnp.float32)],
    )

    return pl.pallas_call(
        _body,
        grid_spec=grid_spec,
        out_shape=jax.ShapeDtypeStruct((B, C), jnp.float32),
        compiler_params=pltpu.CompilerParams(
            dimension_semantics=("arbitrary",),
        ),
    )(s0s, idx3, sc3, X, W1, W2, W3, W4)


# speedup vs baseline: 2.3579x; 2.3579x over previous
import jax
import jax.numpy as jnp
from jax import lax
from jax.experimental import pallas as pl
from jax.experimental.pallas import tpu as pltpu

N = 320000
F_IN = 128
H = 128
C = 64
B = 10000

R = 6400
NBLK = N // R
G = 640
NCH = R // G
W = 32
WF = 128
ACC_ROWS = B + 2 * WF


def _body(s0_ref, idx_ref, sc_ref, x_ref, w1_ref, w2_ref, w3_ref, w4_ref,
          out_ref, acc_ref):
    pid = pl.program_id(0)

    @pl.when(pid == 0)
    def _init():
        acc_ref[...] = jnp.zeros((ACC_ROWS, H), jnp.float32)

    x = x_ref[...].astype(jnp.bfloat16)
    h = jnp.maximum(
        jnp.dot(x, w1_ref[...].astype(jnp.bfloat16),
                preferred_element_type=jnp.float32), 0.0
    ).astype(jnp.bfloat16)

    seg = idx_ref[0]
    sc = sc_ref[0]
    iota = lax.broadcasted_iota(jnp.int32, (W, G), 0)

    ov = jnp.zeros((1, G), jnp.int32)
    for c in range(NCH):
        seg_c = seg[:, c * G:(c + 1) * G]
        sc_c = sc[:, c * G:(c + 1) * G]
        s0c = s0_ref[pid * NCH + c]
        local = seg_c - s0c
        oh = jnp.where(local == iota, sc_c, 0.0).astype(jnp.bfloat16)
        contrib = lax.dot_general(oh, h[c * G:(c + 1) * G, :],
                                  (((1,), (0,)), ((), ())),
                                  preferred_element_type=jnp.float32)
        acc_ref[pl.ds(s0c, W), :] += contrib
        ov = jnp.maximum(ov, local)

    @pl.when(jnp.max(ov) >= W)
    def _fallback():
        pieces = []
        for c in range(NCH):
            seg_c = seg[:, c * G:(c + 1) * G]
            sc_c = sc[:, c * G:(c + 1) * G]
            local = seg_c - s0_ref[pid * NCH + c]
            pieces.append(jnp.where(local >= W, sc_c, 0.0))
        scm = jnp.concatenate(pieces, axis=1)
        s0b = s0_ref[pid * NCH]
        localb = seg - s0b
        nwin = jnp.max(localb) // WF + 1
        iota_f = lax.broadcasted_iota(jnp.int32, (WF, R), 0)

        def win(k, carry):
            base = k * WF
            ohf = jnp.where(localb == base + iota_f, scm,
                            0.0).astype(jnp.bfloat16)
            contrib = lax.dot_general(ohf, h, (((1,), (0,)), ((), ())),
                                      preferred_element_type=jnp.float32)
            acc_ref[pl.ds(s0b + base, WF), :] += contrib
            return carry

        lax.fori_loop(0, nwin, win, 0)

    @pl.when(pid == NBLK - 1)
    def _final():
        w23 = jnp.dot(w2_ref[...].astype(jnp.bfloat16),
                      w3_ref[...].astype(jnp.bfloat16),
                      preferred_element_type=jnp.float32).astype(jnp.bfloat16)
        p = acc_ref[0:B, :].astype(jnp.bfloat16)
        h2 = jnp.maximum(
            jnp.dot(p, w23, preferred_element_type=jnp.float32), 0.0
        ).astype(jnp.bfloat16)
        out_ref[...] = jnp.dot(h2, w4_ref[...].astype(jnp.bfloat16),
                               preferred_element_type=jnp.float32)


def kernel(X, ppr_scores, ppr_idx, W1, W2, W3, W4):
    s0s = ppr_idx[::G]
    idx3 = ppr_idx.reshape(NBLK, 1, R)
    sc3 = ppr_scores.reshape(NBLK, 1, R)

    grid_spec = pltpu.PrefetchScalarGridSpec(
        num_scalar_prefetch=1,
        grid=(NBLK,),
        in_specs=[
            pl.BlockSpec((1, 1, R), lambda i, s0s: (i, 0, 0)),
            pl.BlockSpec((1, 1, R), lambda i, s0s: (i, 0, 0)),
            pl.BlockSpec((R, F_IN), lambda i, s0s: (i, 0)),
            pl.BlockSpec((F_IN, H), lambda i, s0s: (0, 0)),
            pl.BlockSpec((H, H), lambda i, s0s: (0, 0)),
            pl.BlockSpec((H, H), lambda i, s0s: (0, 0)),
            pl.BlockSpec((H, C), lambda i, s0s: (0, 0)),
        ],
        out_specs=pl.BlockSpec((B, C), lambda i, s0s: (0, 0)),
        scratch_shapes=[pltpu.VMEM((ACC_ROWS, H), jnp.float32)],
    )

    return pl.pallas_call(
        _body,
        grid_spec=grid_spec,
        out_shape=jax.ShapeDtypeStruct((B, C), jnp.float32),
        compiler_params=pltpu.CompilerParams(
            dimension_semantics=("arbitrary",),
        ),
    )(s0s, idx3, sc3, X, W1, W2, W3, W4)


# speedup vs baseline: 2.7777x; 1.1780x over previous
import jax
import jax.numpy as jnp
from jax import lax
from jax.experimental import pallas as pl
from jax.experimental.pallas import tpu as pltpu

N = 320000
F_IN = 128
H = 128
C = 64
B = 10000

R = 12800
NBLK = N // R
G = 640
NCH = R // G
W = 32
WF = 128
ACC_ROWS = B + 2 * WF


def _body(s0_ref, idx_ref, sc_ref, x_ref, w1_ref, w2_ref, w3_ref, w4_ref,
          out_ref, acc_ref):
    pid = pl.program_id(0)

    @pl.when(pid == 0)
    def _init():
        acc_ref[...] = jnp.zeros((ACC_ROWS, H), jnp.float32)

    x = x_ref[...].astype(jnp.bfloat16)
    h = jnp.maximum(
        jnp.dot(x, w1_ref[...].astype(jnp.bfloat16),
                preferred_element_type=jnp.float32), 0.0
    ).astype(jnp.bfloat16)

    seg = idx_ref[0]
    sc = sc_ref[0]
    iota = lax.broadcasted_iota(jnp.int32, (W, G), 0)

    ov = jnp.zeros((1, G), jnp.int32)
    for c in range(NCH):
        seg_c = seg[:, c * G:(c + 1) * G]
        sc_c = sc[:, c * G:(c + 1) * G]
        s0c = s0_ref[pid * NCH + c]
        local = seg_c - s0c
        oh = jnp.where(local == iota, sc_c, 0.0).astype(jnp.bfloat16)
        contrib = lax.dot_general(oh, h[c * G:(c + 1) * G, :],
                                  (((1,), (0,)), ((), ())),
                                  preferred_element_type=jnp.float32)
        acc_ref[pl.ds(s0c, W), :] += contrib
        ov = jnp.maximum(ov, local)

    @pl.when(jnp.max(ov) >= W)
    def _fallback():
        pieces = []
        for c in range(NCH):
            seg_c = seg[:, c * G:(c + 1) * G]
            sc_c = sc[:, c * G:(c + 1) * G]
            local = seg_c - s0_ref[pid * NCH + c]
            pieces.append(jnp.where(local >= W, sc_c, 0.0))
        scm = jnp.concatenate(pieces, axis=1)
        s0b = s0_ref[pid * NCH]
        localb = seg - s0b
        nwin = jnp.max(localb) // WF + 1
        iota_f = lax.broadcasted_iota(jnp.int32, (WF, R), 0)

        def win(k, carry):
            base = k * WF
            ohf = jnp.where(localb == base + iota_f, scm,
                            0.0).astype(jnp.bfloat16)
            contrib = lax.dot_general(ohf, h, (((1,), (0,)), ((), ())),
                                      preferred_element_type=jnp.float32)
            acc_ref[pl.ds(s0b + base, WF), :] += contrib
            return carry

        lax.fori_loop(0, nwin, win, 0)

    @pl.when(pid == NBLK - 1)
    def _final():
        w23 = jnp.dot(w2_ref[...].astype(jnp.bfloat16),
                      w3_ref[...].astype(jnp.bfloat16),
                      preferred_element_type=jnp.float32).astype(jnp.bfloat16)
        p = acc_ref[0:B, :].astype(jnp.bfloat16)
        h2 = jnp.maximum(
            jnp.dot(p, w23, preferred_element_type=jnp.float32), 0.0
        ).astype(jnp.bfloat16)
        out_ref[...] = jnp.dot(h2, w4_ref[...].astype(jnp.bfloat16),
                               preferred_element_type=jnp.float32)


def kernel(X, ppr_scores, ppr_idx, W1, W2, W3, W4):
    s0s = ppr_idx[::G]
    idx3 = ppr_idx.reshape(NBLK, 1, R)
    sc3 = ppr_scores.reshape(NBLK, 1, R)

    grid_spec = pltpu.PrefetchScalarGridSpec(
        num_scalar_prefetch=1,
        grid=(NBLK,),
        in_specs=[
            pl.BlockSpec((1, 1, R), lambda i, s0s: (i, 0, 0)),
            pl.BlockSpec((1, 1, R), lambda i, s0s: (i, 0, 0)),
            pl.BlockSpec((R, F_IN), lambda i, s0s: (i, 0)),
            pl.BlockSpec((F_IN, H), lambda i, s0s: (0, 0)),
            pl.BlockSpec((H, H), lambda i, s0s: (0, 0)),
            pl.BlockSpec((H, H), lambda i, s0s: (0, 0)),
            pl.BlockSpec((H, C), lambda i, s0s: (0, 0)),
        ],
        out_specs=pl.BlockSpec((B, C), lambda i, s0s: (0, 0)),
        scratch_shapes=[pltpu.VMEM((ACC_ROWS, H), jnp.float32)],
    )

    return pl.pallas_call(
        _body,
        grid_spec=grid_spec,
        out_shape=jax.ShapeDtypeStruct((B, C), jnp.float32),
        compiler_params=pltpu.CompilerParams(
            dimension_semantics=("arbitrary",),
        ),
    )(s0s, idx3, sc3, X, W1, W2, W3, W4)


# speedup vs baseline: 3.0256x; 1.0893x over previous
import jax
import jax.numpy as jnp
from jax import lax
from jax.experimental import pallas as pl
from jax.experimental.pallas import tpu as pltpu

N = 320000
F_IN = 128
H = 128
C = 64
B = 10000

R = 32000
NBLK = N // R
G = 640
NCH = R // G
W = 32
WF = 128
ACC_ROWS = B + 2 * WF


def _body(s0_ref, idx_ref, sc_ref, x_ref, w1_ref, w2_ref, w3_ref, w4_ref,
          out_ref, acc_ref):
    pid = pl.program_id(0)

    @pl.when(pid == 0)
    def _init():
        acc_ref[...] = jnp.zeros((ACC_ROWS, H), jnp.float32)

    x = x_ref[...].astype(jnp.bfloat16)
    h = jnp.maximum(
        jnp.dot(x, w1_ref[...].astype(jnp.bfloat16),
                preferred_element_type=jnp.float32), 0.0
    ).astype(jnp.bfloat16)

    seg = idx_ref[0]
    sc = sc_ref[0]
    iota = lax.broadcasted_iota(jnp.int32, (W, G), 0)

    ov = jnp.zeros((1, G), jnp.int32)
    for c in range(NCH):
        seg_c = seg[:, c * G:(c + 1) * G]
        sc_c = sc[:, c * G:(c + 1) * G]
        s0c = s0_ref[pid * NCH + c]
        local = seg_c - s0c
        oh = jnp.where(local == iota, sc_c, 0.0).astype(jnp.bfloat16)
        contrib = lax.dot_general(oh, h[c * G:(c + 1) * G, :],
                                  (((1,), (0,)), ((), ())),
                                  preferred_element_type=jnp.float32)
        acc_ref[pl.ds(s0c, W), :] += contrib
        ov = jnp.maximum(ov, local)

    @pl.when(jnp.max(ov) >= W)
    def _fallback():
        pieces = []
        for c in range(NCH):
            seg_c = seg[:, c * G:(c + 1) * G]
            sc_c = sc[:, c * G:(c + 1) * G]
            local = seg_c - s0_ref[pid * NCH + c]
            pieces.append(jnp.where(local >= W, sc_c, 0.0))
        scm = jnp.concatenate(pieces, axis=1)
        s0b = s0_ref[pid * NCH]
        localb = seg - s0b
        nwin = jnp.max(localb) // WF + 1
        iota_f = lax.broadcasted_iota(jnp.int32, (WF, R), 0)

        def win(k, carry):
            base = k * WF
            ohf = jnp.where(localb == base + iota_f, scm,
                            0.0).astype(jnp.bfloat16)
            contrib = lax.dot_general(ohf, h, (((1,), (0,)), ((), ())),
                                      preferred_element_type=jnp.float32)
            acc_ref[pl.ds(s0b + base, WF), :] += contrib
            return carry

        lax.fori_loop(0, nwin, win, 0)

    @pl.when(pid == NBLK - 1)
    def _final():
        w23 = jnp.dot(w2_ref[...].astype(jnp.bfloat16),
                      w3_ref[...].astype(jnp.bfloat16),
                      preferred_element_type=jnp.float32).astype(jnp.bfloat16)
        p = acc_ref[0:B, :].astype(jnp.bfloat16)
        h2 = jnp.maximum(
            jnp.dot(p, w23, preferred_element_type=jnp.float32), 0.0
        ).astype(jnp.bfloat16)
        out_ref[...] = jnp.dot(h2, w4_ref[...].astype(jnp.bfloat16),
                               preferred_element_type=jnp.float32)


def kernel(X, ppr_scores, ppr_idx, W1, W2, W3, W4):
    s0s = ppr_idx[::G]
    idx3 = ppr_idx.reshape(NBLK, 1, R)
    sc3 = ppr_scores.reshape(NBLK, 1, R)

    grid_spec = pltpu.PrefetchScalarGridSpec(
        num_scalar_prefetch=1,
        grid=(NBLK,),
        in_specs=[
            pl.BlockSpec((1, 1, R), lambda i, s0s: (i, 0, 0)),
            pl.BlockSpec((1, 1, R), lambda i, s0s: (i, 0, 0)),
            pl.BlockSpec((R, F_IN), lambda i, s0s: (i, 0)),
            pl.BlockSpec((F_IN, H), lambda i, s0s: (0, 0)),
            pl.BlockSpec((H, H), lambda i, s0s: (0, 0)),
            pl.BlockSpec((H, H), lambda i, s0s: (0, 0)),
            pl.BlockSpec((H, C), lambda i, s0s: (0, 0)),
        ],
        out_specs=pl.BlockSpec((B, C), lambda i, s0s: (0, 0)),
        scratch_shapes=[pltpu.VMEM((ACC_ROWS, H), jnp.float32)],
    )

    return pl.pallas_call(
        _body,
        grid_spec=grid_spec,
        out_shape=jax.ShapeDtypeStruct((B, C), jnp.float32),
        compiler_params=pltpu.CompilerParams(
            dimension_semantics=("arbitrary",),
        ),
    )(s0s, idx3, sc3, X, W1, W2, W3, W4)
